# Initial kernel scaffold; baseline (speedup 1.0000x reference)
import jax
import jax.numpy as jnp
from jax.experimental import pallas as pl


def kernel(edge_index, edge_weight, features, W1, b1, W2, b2):
    raise NotImplementedError("write your pallas kernel here")


# speedup vs baseline: 4.7723x; 4.7723x over previous
import functools

import jax
import jax.numpy as jnp
from jax import lax
from jax.experimental import pallas as pl
from jax.experimental.pallas import tpu as pltpu
from jax.experimental.pallas import tpu_sc as plsc

N_NODES = 10000
D = 128
N_EDGES = 320000
NC = 2
NS = 16
NW = NC * NS
EPW = N_EDGES // NW
CHUNK = 80
NCHUNK = EPW // CHUNK
ROWS_PT = N_NODES // NS


def _full16(v):
  return jnp.full((16,), v, dtype=jnp.int32)


def _spmm_body(src_hbm, dst_hbm, w_hbm, feat_hbm, zeros_hbm, out_hbm,
               src_c, dst_c, w_v, rows_v, xacc, sem):
  cid = lax.axis_index("c")
  sid = lax.axis_index("s")
  wid = cid * NS + sid

  pltpu.sync_copy(zeros_hbm, xacc.at[pl.ds(sid * ROWS_PT, ROWS_PT)])

  pltpu.sync_copy(w_hbm.at[wid], w_v)

  plsc.subcore_barrier()

  def chunk_body(c, carry):
    pltpu.sync_copy(src_hbm.at[wid, c], src_c)
    pltpu.sync_copy(dst_hbm.at[wid, c], dst_c)
    pltpu.async_copy(feat_hbm.at[src_c], rows_v, sem).wait()

    def grp_body(g, carry2):
      wg = w_v[pl.ds(c * CHUNK + g * 16, 16)]
      for j in range(16):
        e = g * 16 + j
        wv = wg[j]
        for q in range(D // 16):
          sl = pl.ds(q * 16, 16)
          rows_v[e, sl] = rows_v[e, sl] * wv
      return carry2

    lax.fori_loop(0, CHUNK // 16, grp_body, 0, unroll=False)

    pltpu.sync_copy(rows_v, xacc.at[dst_c], add=True)
    return carry

  lax.fori_loop(0, NCHUNK, chunk_body, 0, unroll=False)

  plsc.subcore_barrier()

  pltpu.sync_copy(xacc.at[pl.ds(sid * ROWS_PT, ROWS_PT)], out_hbm.at[cid, sid])


@functools.partial(jax.jit, static_argnames=())
def _spmm(src, dst, w, features, zeros):
  mesh = plsc.VectorSubcoreMesh(core_axis_name="c", subcore_axis_name="s")
  k = pl.kernel(
      _spmm_body,
      out_type=jax.ShapeDtypeStruct((NC, NS, ROWS_PT, D), jnp.float32),
      mesh=mesh,
      scratch_types=[
          pltpu.VMEM((CHUNK,), jnp.int32),
          pltpu.VMEM((CHUNK,), jnp.int32),
          pltpu.VMEM((EPW,), jnp.float32),
          pltpu.VMEM((CHUNK, D), jnp.float32),
          pltpu.VMEM_SHARED((N_NODES, D), jnp.float32),
          pltpu.SemaphoreType.DMA,
      ],
  )
  return k(src, dst, w, features, zeros)


def _dense_body(f_ref, xp_ref, w1_ref, w2_ref, b_ref, o_ref):
  x = xp_ref[0] + xp_ref[1]
  f = f_ref[...]
  o_ref[...] = (
      jnp.dot(f + x, w1_ref[...], preferred_element_type=jnp.float32)
      + jnp.dot(x * f, w2_ref[...], preferred_element_type=jnp.float32)
      + b_ref[...]
  )


def _dense(features, xp, W1, W2, b):
  blk = 1000
  grid = N_NODES // blk
  return pl.pallas_call(
      _dense_body,
      grid=(grid,),
      in_specs=[
          pl.BlockSpec((blk, D), lambda i: (i, 0)),
          pl.BlockSpec((NC, blk, D), lambda i: (0, i, 0)),
          pl.BlockSpec((D, D), lambda i: (0, 0)),
          pl.BlockSpec((D, D), lambda i: (0, 0)),
          pl.BlockSpec((1, D), lambda i: (0, 0)),
      ],
      out_specs=pl.BlockSpec((blk, D), lambda i: (i, 0)),
      out_shape=jax.ShapeDtypeStruct((N_NODES, D), jnp.float32),
  )(features, xp, W1, W2, b)


def kernel(edge_index, edge_weight, features, W1, b1, W2, b2):
  src = edge_index[1].reshape(NW, NCHUNK, CHUNK)
  dst = edge_index[0].reshape(NW, NCHUNK, CHUNK)
  w = edge_weight.reshape(NW, EPW)
  zeros = jnp.zeros((ROWS_PT, D), jnp.float32)
  xp = _spmm(src, dst, w, features, zeros).reshape(NC, N_NODES, D)
  b = (b1 + b2).reshape(1, D)
  return _dense(features, xp, W1, W2, b)
